# Initial kernel scaffold; baseline (speedup 1.0000x reference)
import jax
import jax.numpy as jnp
from jax.experimental import pallas as pl


def kernel(feat_q, feat_k):
    raise NotImplementedError("write your pallas kernel here")


# speedup vs baseline: 3.4047x; 3.4047x over previous
import functools

import numpy as np
import jax
import jax.numpy as jnp
from jax import lax
from jax.experimental import pallas as pl
from jax.experimental.pallas import tpu as pltpu
from jax.experimental.pallas import tpu_sc as plsc

_TEMP = 0.07
_NUM_NEG = 128

_NC = 2
_NS = 16
_NW = _NC * _NS

_consts_cache = {}


def _rotl(x, d):
    return ((x << np.uint32(d)) | (x >> np.uint32(32 - d))).astype(np.uint32)


def _threefry2x32(k0, k1, x0, x1):
    rot = [13, 15, 26, 6, 17, 29, 16, 24]
    ks0, ks1 = np.uint32(k0), np.uint32(k1)
    ks2 = np.uint32(ks0 ^ ks1 ^ np.uint32(0x1BD11BDA))
    ks = [ks0, ks1, ks2]
    x0 = (x0 + ks0).astype(np.uint32)
    x1 = (x1 + ks1).astype(np.uint32)
    for i in range(5):
        for r in rot[(i % 2) * 4:(i % 2) * 4 + 4]:
            x0 = (x0 + x1).astype(np.uint32)
            x1 = (_rotl(x1, r) ^ x0).astype(np.uint32)
        x0 = (x0 + ks[(i + 1) % 3]).astype(np.uint32)
        x1 = (x1 + ks[(i + 2) % 3] + np.uint32(i + 1)).astype(np.uint32)
    return x0, x1


def _bits(k0, k1, size):
    counts = np.arange(size, dtype=np.uint32)
    h0, h1 = _threefry2x32(k0, k1, np.zeros(size, np.uint32), counts)
    return h0 ^ h1


def _negative_indices(N, num_neg):
    key_ = (N, num_neg)
    if key_ not in _consts_cache:
        f0, f1 = _threefry2x32(np.uint32(0), np.uint32(0),
                               np.array([0], np.uint32),
                               np.array([123], np.uint32))
        s0, s1 = _threefry2x32(f0[0], f1[0], np.zeros(2, np.uint32),
                               np.arange(2, dtype=np.uint32))
        u = _bits(s0[0], s1[0], N * num_neg)
        v = _bits(s0[1], s1[1], N * num_neg)
        span = np.uint64(N - 1)
        mult = np.uint64((np.uint64(65536 % span) ** 2) % span)
        r = ((u % span).astype(np.uint64) * mult
             + (v % span).astype(np.uint64)) % span
        rand = r.astype(np.int32).reshape(N, num_neg)
        i = np.arange(N, dtype=np.int32)[:, None]
        _consts_cache[key_] = rand + (rand >= i).astype(np.int32)
    return _consts_cache[key_]


def _s1_body(fq_ref, fk_ref, m_ref, pos_ref):
    i = pl.program_id(0)
    q = fq_ref[...]
    qn = q / jnp.maximum(jnp.sqrt(jnp.sum(q * q, axis=1, keepdims=True)), 1e-12)
    k = fk_ref[...]
    kn = k / jnp.maximum(jnp.sqrt(jnp.sum(k * k, axis=0, keepdims=True)), 1e-12)
    m = jnp.dot(qn, kn, preferred_element_type=jnp.float32)
    m_ref[...] = m
    BQ, N = m.shape
    col = lax.broadcasted_iota(jnp.int32, (BQ, N), 1)
    row = lax.broadcasted_iota(jnp.int32, (BQ, N), 0) + i * BQ
    pos_ref[...] = jnp.sum(jnp.where(col == row, m, 0.0), axis=1)


def _similarity(fqm, fkm, bq=512):
    N, C = fqm.shape
    return pl.pallas_call(
        _s1_body,
        grid=(N // bq,),
        in_specs=[
            pl.BlockSpec((bq, C), lambda i: (i, 0)),
            pl.BlockSpec((C, N), lambda i: (0, 0)),
        ],
        out_specs=[
            pl.BlockSpec((bq, N), lambda i: (i, 0)),
            pl.BlockSpec((bq,), lambda i: (i,)),
        ],
        out_shape=[
            jax.ShapeDtypeStruct((N, N), jnp.float32),
            jax.ShapeDtypeStruct((N,), jnp.float32),
        ],
    )(fqm, fkm)


def _sc_neg_expsum(m, idx_flat, N, num_neg):
    rows_per_w = N // _NW
    per_w = rows_per_w * num_neg
    m2 = m.reshape(N * N // 16, 16)

    mesh = plsc.VectorSubcoreMesh(core_axis_name="c", subcore_axis_name="s")

    @functools.partial(
        pl.kernel,
        mesh=mesh,
        compiler_params=pltpu.CompilerParams(needs_layout_passes=False),
        out_type=jax.ShapeDtypeStruct((N * 16,), jnp.float32),
        scratch_types=[
            pltpu.VMEM((per_w,), jnp.int32),
            pltpu.VMEM((N // 16, 16), jnp.float32),
            pltpu.VMEM((rows_per_w * 16,), jnp.float32),
            pltpu.SemaphoreType.DMA,
        ],
    )
    def sc_k(m_hbm, idx_hbm, out_hbm, idx_v, mrow_v, out_v, sem):
        wid = lax.axis_index("s") * _NC + lax.axis_index("c")
        base = wid * per_w
        pltpu.sync_copy(idx_hbm.at[pl.ds(base, per_w)], idx_v)

        def row_body(r, carry):
            g_row = wid * rows_per_w + r
            off = pl.multiple_of(g_row * (N // 16), N // 16)
            pltpu.async_copy(m_hbm.at[pl.ds(off, N // 16)], mrow_v, sem).wait()
            acc = jnp.zeros((16,), jnp.float32)
            for g in range(num_neg // 16):
                ioff = pl.multiple_of(r * num_neg + g * 16, 16)
                cols = idx_v[pl.ds(ioff, 16)]
                vals = plsc.load_gather(mrow_v, [cols >> 4, cols & 15])
                acc = acc + jnp.exp(vals * (1.0 / _TEMP))
            ooff = pl.multiple_of(r * 16, 16)
            out_v[pl.ds(ooff, 16)] = acc
            return carry

        lax.fori_loop(0, rows_per_w, row_body, 0)
        pltpu.sync_copy(out_v, out_hbm.at[pl.ds(wid * rows_per_w * 16,
                                                rows_per_w * 16)])

    return sc_k(m2, idx_flat)


def _s3_body(pos_ref, p_ref, out_ref):
    pos = pos_ref[...]
    s = jnp.sum(p_ref[...], axis=1)
    t = pos * (1.0 / _TEMP)
    out_ref[...] = jnp.reshape(jnp.mean(jnp.log(jnp.exp(t) + s) - t), (1, 1))


def _finalize(pos, partial):
    N = pos.shape[0]
    return pl.pallas_call(
        _s3_body,
        in_specs=[
            pl.BlockSpec((N,), lambda: (0,)),
            pl.BlockSpec((N, 16), lambda: (0, 0)),
        ],
        out_specs=pl.BlockSpec((1, 1), lambda: (0, 0)),
        out_shape=jax.ShapeDtypeStruct((1, 1), jnp.float32),
    )(pos, partial)


def kernel(feat_q, feat_k):
    B, C, H, W = feat_q.shape
    HW = H * W
    N = B * HW
    fq3 = feat_q.reshape(B, C, HW)
    fk3 = feat_k.reshape(B, C, HW)
    fqm = fq3.transpose(0, 2, 1).reshape(N, C)
    fkm = fk3.transpose(1, 0, 2).reshape(C, N)

    m, pos = _similarity(fqm, fkm)

    idx = _negative_indices(N, _NUM_NEG)
    partial_flat = _sc_neg_expsum(m, jnp.asarray(idx.reshape(-1)), N, _NUM_NEG)

    loss = _finalize(pos, partial_flat.reshape(N, 16))
    return loss[0, 0]


# speedup vs baseline: 20.5951x; 6.0490x over previous
import functools

import numpy as np
import jax
import jax.numpy as jnp
from jax import lax
from jax.experimental import pallas as pl
from jax.experimental.pallas import tpu as pltpu
from jax.experimental.pallas import tpu_sc as plsc

_TEMP = 0.07
_NUM_NEG = 128

_NC = 2
_NS = 16
_NW = _NC * _NS

_consts_cache = {}


def _rotl(x, d):
    return ((x << np.uint32(d)) | (x >> np.uint32(32 - d))).astype(np.uint32)


def _threefry2x32(k0, k1, x0, x1):
    rot = [13, 15, 26, 6, 17, 29, 16, 24]
    ks0, ks1 = np.uint32(k0), np.uint32(k1)
    ks2 = np.uint32(ks0 ^ ks1 ^ np.uint32(0x1BD11BDA))
    ks = [ks0, ks1, ks2]
    x0 = (x0 + ks0).astype(np.uint32)
    x1 = (x1 + ks1).astype(np.uint32)
    for i in range(5):
        for r in rot[(i % 2) * 4:(i % 2) * 4 + 4]:
            x0 = (x0 + x1).astype(np.uint32)
            x1 = (_rotl(x1, r) ^ x0).astype(np.uint32)
        x0 = (x0 + ks[(i + 1) % 3]).astype(np.uint32)
        x1 = (x1 + ks[(i + 2) % 3] + np.uint32(i + 1)).astype(np.uint32)
    return x0, x1


def _bits(k0, k1, size):
    counts = np.arange(size, dtype=np.uint32)
    h0, h1 = _threefry2x32(k0, k1, np.zeros(size, np.uint32), counts)
    return h0 ^ h1


def _negative_indices(N, num_neg):
    key_ = (N, num_neg)
    if key_ not in _consts_cache:
        f0, f1 = _threefry2x32(np.uint32(0), np.uint32(0),
                               np.array([0], np.uint32),
                               np.array([123], np.uint32))
        s0, s1 = _threefry2x32(f0[0], f1[0], np.zeros(2, np.uint32),
                               np.arange(2, dtype=np.uint32))
        u = _bits(s0[0], s1[0], N * num_neg)
        v = _bits(s0[1], s1[1], N * num_neg)
        span = np.uint64(N - 1)
        mult = np.uint64((np.uint64(65536 % span) ** 2) % span)
        r = ((u % span).astype(np.uint64) * mult
             + (v % span).astype(np.uint64)) % span
        rand = r.astype(np.int32).reshape(N, num_neg)
        i = np.arange(N, dtype=np.int32)[:, None]
        _consts_cache[key_] = rand + (rand >= i).astype(np.int32)
    return _consts_cache[key_]


def _s1_body(fq_ref, fk_ref, m_ref, pos_ref):
    i = pl.program_id(0)
    q = fq_ref[...]
    qn = q / jnp.maximum(jnp.sqrt(jnp.sum(q * q, axis=1, keepdims=True)), 1e-12)
    k = fk_ref[...]
    kss = jnp.sum(k * k, axis=0, keepdims=True)
    kinv = 1.0 / jnp.maximum(jnp.sqrt(kss), 1e-12)
    BQ = q.shape[0]
    N = k.shape[1]
    acc_pos = jnp.zeros((BQ,), jnp.float32)
    row = lax.broadcasted_iota(jnp.int32, (BQ, 128), 0) + i * BQ
    for t in range(N // 128):
        kn_t = k[:, t * 128:(t + 1) * 128] * kinv[:, t * 128:(t + 1) * 128]
        m_t = jnp.dot(qn, kn_t, preferred_element_type=jnp.float32)
        m_ref[t * BQ:(t + 1) * BQ, :] = m_t
        col = lax.broadcasted_iota(jnp.int32, (BQ, 128), 1) + t * 128
        acc_pos = acc_pos + jnp.sum(jnp.where(col == row, m_t, 0.0), axis=1)
    pos_ref[...] = acc_pos


def _similarity(fqm, fkm, bq=512):
    N, C = fqm.shape
    nt = N // 128
    return pl.pallas_call(
        _s1_body,
        grid=(N // bq,),
        in_specs=[
            pl.BlockSpec((bq, C), lambda i: (i, 0)),
            pl.BlockSpec((C, N), lambda i: (0, 0)),
        ],
        out_specs=[
            pl.BlockSpec((bq * nt, 128), lambda i: (i, 0)),
            pl.BlockSpec((bq,), lambda i: (i,)),
        ],
        out_shape=[
            jax.ShapeDtypeStruct((N * nt, 128), jnp.float32),
            jax.ShapeDtypeStruct((N,), jnp.float32),
        ],
    )(fqm, fkm)


def _chunk_row_constants(N, num_neg, bq):
    key_ = ("crow", N, num_neg, bq)
    if key_ not in _consts_cache:
        nt = N // 128
        n = np.arange(N, dtype=np.int32)[:, None]
        t = np.arange(nt, dtype=np.int32)[None, :]
        crow = (n // bq) * (bq * nt) + t * bq + (n % bq)
        cidx = _negative_indices(N, num_neg)
        _consts_cache[key_] = (crow, cidx.reshape(-1))
    return _consts_cache[key_]


def _sc_neg_expsum(m_out, N, num_neg, bq):
    rows_per_w = N // _NW
    nt = N // 128
    qb = 8
    nblk = rows_per_w // qb
    crow, cidx_flat = _chunk_row_constants(N, num_neg, bq)

    mesh = plsc.VectorSubcoreMesh(core_axis_name="c", subcore_axis_name="s")

    @functools.partial(
        pl.kernel,
        mesh=mesh,
        compiler_params=pltpu.CompilerParams(needs_layout_passes=False),
        out_type=jax.ShapeDtypeStruct((N * 16,), jnp.float32),
        scratch_types=[
            pltpu.VMEM((rows_per_w, nt), jnp.int32),
            pltpu.VMEM((rows_per_w * num_neg,), jnp.int32),
            pltpu.VMEM((qb * nt, 128), jnp.float32),
            pltpu.VMEM((qb * nt, 128), jnp.float32),
            pltpu.VMEM((rows_per_w * 16,), jnp.float32),
            pltpu.SemaphoreType.DMA,
            pltpu.SemaphoreType.DMA,
        ],
    )
    def sc_k(tab_hbm, crow_hbm, cidx_hbm, out_hbm,
             crow_v, cidx_v, buf_a, buf_b, out_v, sem_a, sem_b):
        wid = lax.axis_index("s") * _NC + lax.axis_index("c")
        pltpu.sync_copy(crow_hbm.at[pl.ds(wid * rows_per_w, rows_per_w)],
                        crow_v)
        pltpu.sync_copy(cidx_hbm.at[pl.ds(wid * rows_per_w * num_neg,
                                          rows_per_w * num_neg)], cidx_v)

        bufs = (buf_a, buf_b)
        sems = (sem_a, sem_b)

        def fire(blk):
            buf, sem = bufs[blk % 2], sems[blk % 2]
            cps = []
            for i in range(qb):
                r = blk * qb + i
                cps.append(pltpu.async_copy(
                    tab_hbm.at[crow_v.at[r]],
                    buf.at[pl.ds(i * nt, nt)], sem))
            return cps

        def compute(blk):
            buf = bufs[blk % 2]

            def row_body(r_loc, carry):
                acc = jnp.zeros((16,), jnp.float32)
                base = pl.multiple_of((blk * qb + r_loc) * num_neg, num_neg)
                for g in range(num_neg // 16):
                    cols = cidx_v[pl.ds(base + g * 16, 16)]
                    vals = plsc.load_gather(
                        buf, [(cols >> 7) + r_loc * nt, cols & 127])
                    acc = acc + jnp.exp(vals * (1.0 / _TEMP))
                ooff = pl.multiple_of((blk * qb + r_loc) * 16, 16)
                out_v[pl.ds(ooff, 16)] = acc
                return carry

            lax.fori_loop(0, qb, row_body, 0)

        pending = fire(0)
        for blk in range(nblk):
            nxt = fire(blk + 1) if blk + 1 < nblk else []
            for cp in pending:
                cp.wait()
            compute(blk)
            pending = nxt

        pltpu.sync_copy(out_v, out_hbm.at[pl.ds(wid * rows_per_w * 16,
                                                rows_per_w * 16)])

    return sc_k(m_out, jnp.asarray(crow), jnp.asarray(cidx_flat))


def _s3_body(pos_ref, p_ref, out_ref):
    pos = pos_ref[...]
    s = jnp.sum(p_ref[...], axis=1)
    t = pos * (1.0 / _TEMP)
    out_ref[...] = jnp.reshape(jnp.mean(jnp.log(jnp.exp(t) + s) - t), (1, 1))


def _finalize(pos, partial):
    N = pos.shape[0]
    return pl.pallas_call(
        _s3_body,
        in_specs=[
            pl.BlockSpec((N,), lambda: (0,)),
            pl.BlockSpec((N, 16), lambda: (0, 0)),
        ],
        out_specs=pl.BlockSpec((1, 1), lambda: (0, 0)),
        out_shape=jax.ShapeDtypeStruct((1, 1), jnp.float32),
    )(pos, partial)


def kernel(feat_q, feat_k):
    B, C, H, W = feat_q.shape
    HW = H * W
    N = B * HW
    fq3 = feat_q.reshape(B, C, HW)
    fk3 = feat_k.reshape(B, C, HW)
    fqm = fq3.transpose(0, 2, 1).reshape(N, C)
    fkm = fk3.transpose(1, 0, 2).reshape(C, N)

    bq = 512
    m_out, pos = _similarity(fqm, fkm, bq=bq)
    partial_flat = _sc_neg_expsum(m_out, N, _NUM_NEG, bq)
    loss = _finalize(pos, partial_flat.reshape(N, 16))
    return loss[0, 0]


# speedup vs baseline: 21.8628x; 1.0616x over previous
import functools

import numpy as np
import jax
import jax.numpy as jnp
from jax import lax
from jax.experimental import pallas as pl
from jax.experimental.pallas import tpu as pltpu
from jax.experimental.pallas import tpu_sc as plsc

_TEMP = 0.07
_NUM_NEG = 128

_NC = 2
_NS = 16
_NW = _NC * _NS

_consts_cache = {}


def _rotl(x, d):
    return ((x << np.uint32(d)) | (x >> np.uint32(32 - d))).astype(np.uint32)


def _threefry2x32(k0, k1, x0, x1):
    rot = [13, 15, 26, 6, 17, 29, 16, 24]
    ks0, ks1 = np.uint32(k0), np.uint32(k1)
    ks2 = np.uint32(ks0 ^ ks1 ^ np.uint32(0x1BD11BDA))
    ks = [ks0, ks1, ks2]
    x0 = (x0 + ks0).astype(np.uint32)
    x1 = (x1 + ks1).astype(np.uint32)
    for i in range(5):
        for r in rot[(i % 2) * 4:(i % 2) * 4 + 4]:
            x0 = (x0 + x1).astype(np.uint32)
            x1 = (_rotl(x1, r) ^ x0).astype(np.uint32)
        x0 = (x0 + ks[(i + 1) % 3]).astype(np.uint32)
        x1 = (x1 + ks[(i + 2) % 3] + np.uint32(i + 1)).astype(np.uint32)
    return x0, x1


def _bits(k0, k1, size):
    counts = np.arange(size, dtype=np.uint32)
    h0, h1 = _threefry2x32(k0, k1, np.zeros(size, np.uint32), counts)
    return h0 ^ h1


def _negative_indices(N, num_neg):
    key_ = (N, num_neg)
    if key_ not in _consts_cache:
        f0, f1 = _threefry2x32(np.uint32(0), np.uint32(0),
                               np.array([0], np.uint32),
                               np.array([123], np.uint32))
        s0, s1 = _threefry2x32(f0[0], f1[0], np.zeros(2, np.uint32),
                               np.arange(2, dtype=np.uint32))
        u = _bits(s0[0], s1[0], N * num_neg)
        v = _bits(s0[1], s1[1], N * num_neg)
        span = np.uint64(N - 1)
        mult = np.uint64((np.uint64(65536 % span) ** 2) % span)
        r = ((u % span).astype(np.uint64) * mult
             + (v % span).astype(np.uint64)) % span
        rand = r.astype(np.int32).reshape(N, num_neg)
        i = np.arange(N, dtype=np.int32)[:, None]
        _consts_cache[key_] = rand + (rand >= i).astype(np.int32)
    return _consts_cache[key_]


def _s1_body(fq_ref, fk_ref, m_ref, pos_ref):
    i = pl.program_id(0)
    qT = fq_ref[0]
    qss = jnp.sum(qT * qT, axis=0, keepdims=True)
    qnT = qT / jnp.maximum(jnp.sqrt(qss), 1e-12)
    C, BQ = qT.shape
    HW = fk_ref.shape[2]
    NT = (fk_ref.shape[0] * HW) // 128
    acc_pos = jnp.zeros((BQ,), jnp.float32)
    row = lax.broadcasted_iota(jnp.int32, (BQ, 128), 0) + i * BQ
    for t in range(NT):
        b2 = (t * 128) // HW
        o = (t * 128) % HW
        k_t = fk_ref[b2][:, o:o + 128]
        kss = jnp.sum(k_t * k_t, axis=0, keepdims=True)
        kn_t = k_t / jnp.maximum(jnp.sqrt(kss), 1e-12)
        m_t = lax.dot_general(qnT, kn_t, (((0,), (0,)), ((), ())),
                              preferred_element_type=jnp.float32)
        m_ref[t * BQ:(t + 1) * BQ, :] = m_t
        col = lax.broadcasted_iota(jnp.int32, (BQ, 128), 1) + t * 128
        acc_pos = acc_pos + jnp.sum(jnp.where(col == row, m_t, 0.0), axis=1)
    pos_ref[...] = acc_pos


def _similarity(fq3, fk3, bq=512):
    B, C, HW = fq3.shape
    N = B * HW
    nt = N // 128
    nb = HW // bq
    return pl.pallas_call(
        _s1_body,
        grid=(N // bq,),
        in_specs=[
            pl.BlockSpec((1, C, bq), lambda i: (i // nb, 0, i % nb)),
            pl.BlockSpec((B, C, HW), lambda i: (0, 0, 0)),
        ],
        out_specs=[
            pl.BlockSpec((bq * nt, 128), lambda i: (i, 0)),
            pl.BlockSpec((bq,), lambda i: (i,)),
        ],
        out_shape=[
            jax.ShapeDtypeStruct((N * nt, 128), jnp.float32),
            jax.ShapeDtypeStruct((N,), jnp.float32),
        ],
    )(fq3, fk3)


def _chunk_row_constants(N, num_neg, bq):
    key_ = ("crow", N, num_neg, bq)
    if key_ not in _consts_cache:
        nt = N // 128
        n = np.arange(N, dtype=np.int32)[:, None]
        t = np.arange(nt, dtype=np.int32)[None, :]
        crow = (n // bq) * (bq * nt) + t * bq + (n % bq)
        cidx = _negative_indices(N, num_neg)
        _consts_cache[key_] = (crow, cidx.reshape(-1))
    return _consts_cache[key_]


def _sc_neg_expsum(m_out, N, num_neg, bq):
    rows_per_w = N // _NW
    nt = N // 128
    qb = 8
    nblk = rows_per_w // qb
    crow, cidx_flat = _chunk_row_constants(N, num_neg, bq)

    mesh = plsc.VectorSubcoreMesh(core_axis_name="c", subcore_axis_name="s")

    @functools.partial(
        pl.kernel,
        mesh=mesh,
        compiler_params=pltpu.CompilerParams(needs_layout_passes=False),
        out_type=jax.ShapeDtypeStruct((N * 16,), jnp.float32),
        scratch_types=[
            pltpu.VMEM((rows_per_w, nt), jnp.int32),
            pltpu.VMEM((rows_per_w * num_neg,), jnp.int32),
            pltpu.VMEM((qb * nt, 128), jnp.float32),
            pltpu.VMEM((qb * nt, 128), jnp.float32),
            pltpu.VMEM((rows_per_w * 16,), jnp.float32),
            pltpu.SemaphoreType.DMA,
            pltpu.SemaphoreType.DMA,
        ],
    )
    def sc_k(tab_hbm, crow_hbm, cidx_hbm, out_hbm,
             crow_v, cidx_v, buf_a, buf_b, out_v, sem_a, sem_b):
        wid = lax.axis_index("s") * _NC + lax.axis_index("c")
        pltpu.sync_copy(crow_hbm.at[pl.ds(wid * rows_per_w, rows_per_w)],
                        crow_v)
        pltpu.sync_copy(cidx_hbm.at[pl.ds(wid * rows_per_w * num_neg,
                                          rows_per_w * num_neg)], cidx_v)

        bufs = (buf_a, buf_b)
        sems = (sem_a, sem_b)

        def fire(blk):
            buf, sem = bufs[blk % 2], sems[blk % 2]
            cps = []
            for i in range(qb):
                r = blk * qb + i
                cps.append(pltpu.async_copy(
                    tab_hbm.at[crow_v.at[r]],
                    buf.at[pl.ds(i * nt, nt)], sem))
            return cps

        def compute(blk):
            buf = bufs[blk % 2]

            def row_body(r_loc, carry):
                n_glob = wid * rows_per_w + blk * qb + r_loc
                dcols = jnp.full((16,), 0, jnp.int32) + n_glob
                dvals = plsc.load_gather(
                    buf, [(dcols >> 7) + r_loc * nt, dcols & 127])
                lane0 = lax.iota(jnp.int32, 16) == 0
                acc = jnp.where(lane0, jnp.exp(dvals * (1.0 / _TEMP)), 0.0)
                base = pl.multiple_of((blk * qb + r_loc) * num_neg, num_neg)
                for g in range(num_neg // 16):
                    cols = cidx_v[pl.ds(base + g * 16, 16)]
                    vals = plsc.load_gather(
                        buf, [(cols >> 7) + r_loc * nt, cols & 127])
                    acc = acc + jnp.exp(vals * (1.0 / _TEMP))
                ooff = pl.multiple_of((blk * qb + r_loc) * 16, 16)
                out_v[pl.ds(ooff, 16)] = acc
                return carry

            lax.fori_loop(0, qb, row_body, 0)

        pending = fire(0)
        for blk in range(nblk):
            nxt = fire(blk + 1) if blk + 1 < nblk else []
            for cp in pending:
                cp.wait()
            compute(blk)
            pending = nxt

        pltpu.sync_copy(out_v, out_hbm.at[pl.ds(wid * rows_per_w * 16,
                                                rows_per_w * 16)])

    return sc_k(m_out, jnp.asarray(crow), jnp.asarray(cidx_flat))


def _s3_body(N, pos_ref, p_ref, out_ref):
    p = p_ref[...]
    gr = lax.broadcasted_iota(jnp.int32, (128, 8), 0) // 16
    gc = lax.broadcasted_iota(jnp.int32, (128, 8), 1)
    G = jnp.where(gr == gc, 1.0, 0.0)
    s8 = jnp.dot(p, G, preferred_element_type=jnp.float32)
    ts = jnp.sum(pos_ref[...]) * (1.0 / _TEMP)
    out_ref[...] = jnp.reshape((jnp.sum(jnp.log(s8)) - ts) / N, (1, 1))


def _finalize(pos32, p512, N):
    return pl.pallas_call(
        functools.partial(_s3_body, N),
        in_specs=[
            pl.BlockSpec(pos32.shape, lambda: (0, 0)),
            pl.BlockSpec(p512.shape, lambda: (0, 0)),
        ],
        out_specs=pl.BlockSpec((1, 1), lambda: (0, 0)),
        out_shape=jax.ShapeDtypeStruct((1, 1), jnp.float32),
    )(pos32, p512)


def kernel(feat_q, feat_k):
    B, C, H, W = feat_q.shape
    HW = H * W
    N = B * HW
    fq3 = feat_q.reshape(B, C, HW)
    fk3 = feat_k.reshape(B, C, HW)

    bq = 512
    m_out, pos = _similarity(fq3, fk3, bq=bq)
    partial_flat = _sc_neg_expsum(m_out, N, _NUM_NEG, bq)
    loss = _finalize(pos.reshape(N // 128, 128),
                     partial_flat.reshape(N // 8, 128), N)
    return loss[0, 0]


# speedup vs baseline: 25.8840x; 1.1839x over previous
import functools

import numpy as np
import jax
import jax.numpy as jnp
from jax import lax
from jax.experimental import pallas as pl
from jax.experimental.pallas import tpu as pltpu
from jax.experimental.pallas import tpu_sc as plsc

_TEMP = 0.07
_NUM_NEG = 128

_NC = 2
_NS = 16
_NW = _NC * _NS

_consts_cache = {}


def _rotl(x, d):
    return ((x << np.uint32(d)) | (x >> np.uint32(32 - d))).astype(np.uint32)


def _threefry2x32(k0, k1, x0, x1):
    rot = [13, 15, 26, 6, 17, 29, 16, 24]
    ks0, ks1 = np.uint32(k0), np.uint32(k1)
    ks2 = np.uint32(ks0 ^ ks1 ^ np.uint32(0x1BD11BDA))
    ks = [ks0, ks1, ks2]
    x0 = (x0 + ks0).astype(np.uint32)
    x1 = (x1 + ks1).astype(np.uint32)
    for i in range(5):
        for r in rot[(i % 2) * 4:(i % 2) * 4 + 4]:
            x0 = (x0 + x1).astype(np.uint32)
            x1 = (_rotl(x1, r) ^ x0).astype(np.uint32)
        x0 = (x0 + ks[(i + 1) % 3]).astype(np.uint32)
        x1 = (x1 + ks[(i + 2) % 3] + np.uint32(i + 1)).astype(np.uint32)
    return x0, x1


def _bits(k0, k1, size):
    counts = np.arange(size, dtype=np.uint32)
    h0, h1 = _threefry2x32(k0, k1, np.zeros(size, np.uint32), counts)
    return h0 ^ h1


def _negative_indices(N, num_neg):
    key_ = (N, num_neg)
    if key_ not in _consts_cache:
        f0, f1 = _threefry2x32(np.uint32(0), np.uint32(0),
                               np.array([0], np.uint32),
                               np.array([123], np.uint32))
        s0, s1 = _threefry2x32(f0[0], f1[0], np.zeros(2, np.uint32),
                               np.arange(2, dtype=np.uint32))
        u = _bits(s0[0], s1[0], N * num_neg)
        v = _bits(s0[1], s1[1], N * num_neg)
        span = np.uint64(N - 1)
        mult = np.uint64((np.uint64(65536 % span) ** 2) % span)
        r = ((u % span).astype(np.uint64) * mult
             + (v % span).astype(np.uint64)) % span
        rand = r.astype(np.int32).reshape(N, num_neg)
        i = np.arange(N, dtype=np.int32)[:, None]
        _consts_cache[key_] = rand + (rand >= i).astype(np.int32)
    return _consts_cache[key_]


def _s1_body(fq_ref, fk_ref, m_ref, pos_ref):
    i = pl.program_id(0)
    qT = fq_ref[0]
    qss = jnp.sum(qT * qT, axis=0, keepdims=True)
    qnT = qT / jnp.maximum(jnp.sqrt(qss), 1e-12)
    C, BQ = qT.shape
    HW = fk_ref.shape[2]
    NT = (fk_ref.shape[0] * HW) // 128
    acc_pos = jnp.zeros((BQ,), jnp.float32)
    row = lax.broadcasted_iota(jnp.int32, (BQ, 128), 0) + i * BQ
    for t in range(NT):
        b2 = (t * 128) // HW
        o = (t * 128) % HW
        k_t = fk_ref[b2][:, o:o + 128]
        kss = jnp.sum(k_t * k_t, axis=0, keepdims=True)
        kn_t = k_t / jnp.maximum(jnp.sqrt(kss), 1e-12)
        m_t = lax.dot_general(qnT, kn_t, (((0,), (0,)), ((), ())),
                              preferred_element_type=jnp.float32)
        m_ref[t * BQ:(t + 1) * BQ, :] = m_t.astype(jnp.bfloat16)
        col = lax.broadcasted_iota(jnp.int32, (BQ, 128), 1) + t * 128
        acc_pos = acc_pos + jnp.sum(jnp.where(col == row, m_t, 0.0), axis=1)
    pos_ref[...] = acc_pos


def _similarity(fq3, fk3, bq=512):
    B, C, HW = fq3.shape
    N = B * HW
    nt = N // 128
    nb = HW // bq
    return pl.pallas_call(
        _s1_body,
        grid=(N // bq,),
        in_specs=[
            pl.BlockSpec((1, C, bq), lambda i: (i // nb, 0, i % nb)),
            pl.BlockSpec((B, C, HW), lambda i: (0, 0, 0)),
        ],
        out_specs=[
            pl.BlockSpec((bq * nt, 128), lambda i: (i, 0)),
            pl.BlockSpec((bq,), lambda i: (i,)),
        ],
        out_shape=[
            jax.ShapeDtypeStruct((N * nt, 128), jnp.bfloat16),
            jax.ShapeDtypeStruct((N,), jnp.float32),
        ],
    )(fq3, fk3)


def _chunk_row_constants(N, num_neg, bq):
    key_ = ("crow", N, num_neg, bq)
    if key_ not in _consts_cache:
        nt = N // 128
        hb = bq // 2
        p = np.arange(N // 2, dtype=np.int32)[:, None]
        t = np.arange(nt, dtype=np.int32)[None, :]
        crow = (p // hb) * (hb * nt) + t * hb + (p % hb)
        cidx = _negative_indices(N, num_neg)
        _consts_cache[key_] = (crow, cidx.reshape(-1))
    return _consts_cache[key_]


def _sc_neg_expsum(m_out, N, num_neg, bq):
    rows_per_w = N // _NW
    nt = N // 128
    pairs_per_w = rows_per_w // 2
    pb = 4
    nblk = pairs_per_w // pb
    crow, cidx_flat = _chunk_row_constants(N, num_neg, bq)

    mesh = plsc.VectorSubcoreMesh(core_axis_name="c", subcore_axis_name="s")

    @functools.partial(
        pl.kernel,
        mesh=mesh,
        compiler_params=pltpu.CompilerParams(needs_layout_passes=False),
        out_type=jax.ShapeDtypeStruct((N * 16,), jnp.float32),
        scratch_types=[
            pltpu.VMEM((pairs_per_w, nt), jnp.int32),
            pltpu.VMEM((rows_per_w * num_neg,), jnp.int32),
            pltpu.VMEM((pb * nt, 128), jnp.int32),
            pltpu.VMEM((pb * nt, 128), jnp.int32),
            pltpu.VMEM((rows_per_w * 16,), jnp.float32),
            pltpu.SemaphoreType.DMA,
            pltpu.SemaphoreType.DMA,
        ],
    )
    def sc_k(tab_hbm, crow_hbm, cidx_hbm, out_hbm,
             crow_v, cidx_v, buf_a, buf_b, out_v, sem_a, sem_b):
        wid = lax.axis_index("s") * _NC + lax.axis_index("c")
        tab32 = tab_hbm.bitcast(jnp.int32)
        pltpu.sync_copy(crow_hbm.at[pl.ds(wid * pairs_per_w, pairs_per_w)],
                        crow_v)
        pltpu.sync_copy(cidx_hbm.at[pl.ds(wid * rows_per_w * num_neg,
                                          rows_per_w * num_neg)], cidx_v)

        bufs = (buf_a, buf_b)
        sems = (sem_a, sem_b)

        def fire(blk):
            buf, sem = bufs[blk % 2], sems[blk % 2]
            cps = []
            for i in range(pb):
                r = blk * pb + i
                cps.append(pltpu.async_copy(
                    tab32.at[crow_v.at[r]],
                    buf.at[pl.ds(i * nt, nt)], sem))
            return cps

        def expv(buf, rows, cols, shift):
            bits = plsc.load_gather(buf, [rows, cols & 127])
            v16 = (bits >> shift) & jnp.int32(0xFFFF)
            f = plsc.bitcast(v16 << 16, jnp.float32)
            return jnp.exp(f * (1.0 / _TEMP))

        def compute(blk):
            buf = bufs[blk % 2]

            def row_body(q_loc, carry):
                pair_slot = q_loc >> 1
                shift = (q_loc & 1) * 16
                n_glob = wid * rows_per_w + blk * 2 * pb + q_loc
                dcols = jnp.full((16,), 0, jnp.int32) + n_glob
                lane0 = lax.iota(jnp.int32, 16) == 0
                de = expv(buf, (dcols >> 7) + pair_slot * nt, dcols, shift)
                acc = jnp.where(lane0, de, 0.0)
                base = pl.multiple_of(
                    (blk * 2 * pb + q_loc) * num_neg, num_neg)
                for g in range(num_neg // 16):
                    cols = cidx_v[pl.ds(base + g * 16, 16)]
                    acc = acc + expv(
                        buf, (cols >> 7) + pair_slot * nt, cols, shift)
                ooff = pl.multiple_of((blk * 2 * pb + q_loc) * 16, 16)
                out_v[pl.ds(ooff, 16)] = acc
                return carry

            lax.fori_loop(0, 2 * pb, row_body, 0)

        pending = fire(0)
        for blk in range(nblk):
            nxt = fire(blk + 1) if blk + 1 < nblk else []
            for cp in pending:
                cp.wait()
            compute(blk)
            pending = nxt

        pltpu.sync_copy(out_v, out_hbm.at[pl.ds(wid * rows_per_w * 16,
                                                rows_per_w * 16)])

    return sc_k(m_out, jnp.asarray(crow), jnp.asarray(cidx_flat))


def _s3_body(N, pos_ref, p_ref, out_ref):
    p = p_ref[...]
    gr = lax.broadcasted_iota(jnp.int32, (128, 8), 0) // 16
    gc = lax.broadcasted_iota(jnp.int32, (128, 8), 1)
    G = jnp.where(gr == gc, 1.0, 0.0)
    s8 = jnp.dot(p, G, preferred_element_type=jnp.float32)
    ts = jnp.sum(pos_ref[...]) * (1.0 / _TEMP)
    out_ref[...] = jnp.reshape((jnp.sum(jnp.log(s8)) - ts) / N, (1, 1))


def _finalize(pos32, p512, N):
    return pl.pallas_call(
        functools.partial(_s3_body, N),
        in_specs=[
            pl.BlockSpec(pos32.shape, lambda: (0, 0)),
            pl.BlockSpec(p512.shape, lambda: (0, 0)),
        ],
        out_specs=pl.BlockSpec((1, 1), lambda: (0, 0)),
        out_shape=jax.ShapeDtypeStruct((1, 1), jnp.float32),
    )(pos32, p512)


def kernel(feat_q, feat_k):
    B, C, H, W = feat_q.shape
    HW = H * W
    N = B * HW
    fq3 = feat_q.reshape(B, C, HW)
    fk3 = feat_k.reshape(B, C, HW)

    bq = 512
    m_out, pos = _similarity(fq3, fk3, bq=bq)
    partial_flat = _sc_neg_expsum(m_out, N, _NUM_NEG, bq)
    loss = _finalize(pos.reshape(N // 128, 128),
                     partial_flat.reshape(N // 8, 128), N)
    return loss[0, 0]


# speedup vs baseline: 25.8938x; 1.0004x over previous
import functools

import numpy as np
import jax
import jax.numpy as jnp
from jax import lax
from jax.experimental import pallas as pl
from jax.experimental.pallas import tpu as pltpu
from jax.experimental.pallas import tpu_sc as plsc

_TEMP = 0.07
_NUM_NEG = 128

_NC = 2
_NS = 16
_NW = _NC * _NS

_consts_cache = {}


def _rotl(x, d):
    return ((x << np.uint32(d)) | (x >> np.uint32(32 - d))).astype(np.uint32)


def _threefry2x32(k0, k1, x0, x1):
    rot = [13, 15, 26, 6, 17, 29, 16, 24]
    ks0, ks1 = np.uint32(k0), np.uint32(k1)
    ks2 = np.uint32(ks0 ^ ks1 ^ np.uint32(0x1BD11BDA))
    ks = [ks0, ks1, ks2]
    x0 = (x0 + ks0).astype(np.uint32)
    x1 = (x1 + ks1).astype(np.uint32)
    for i in range(5):
        for r in rot[(i % 2) * 4:(i % 2) * 4 + 4]:
            x0 = (x0 + x1).astype(np.uint32)
            x1 = (_rotl(x1, r) ^ x0).astype(np.uint32)
        x0 = (x0 + ks[(i + 1) % 3]).astype(np.uint32)
        x1 = (x1 + ks[(i + 2) % 3] + np.uint32(i + 1)).astype(np.uint32)
    return x0, x1


def _bits(k0, k1, size):
    counts = np.arange(size, dtype=np.uint32)
    h0, h1 = _threefry2x32(k0, k1, np.zeros(size, np.uint32), counts)
    return h0 ^ h1


def _negative_indices(N, num_neg):
    key_ = (N, num_neg)
    if key_ not in _consts_cache:
        f0, f1 = _threefry2x32(np.uint32(0), np.uint32(0),
                               np.array([0], np.uint32),
                               np.array([123], np.uint32))
        s0, s1 = _threefry2x32(f0[0], f1[0], np.zeros(2, np.uint32),
                               np.arange(2, dtype=np.uint32))
        u = _bits(s0[0], s1[0], N * num_neg)
        v = _bits(s0[1], s1[1], N * num_neg)
        span = np.uint64(N - 1)
        mult = np.uint64((np.uint64(65536 % span) ** 2) % span)
        r = ((u % span).astype(np.uint64) * mult
             + (v % span).astype(np.uint64)) % span
        rand = r.astype(np.int32).reshape(N, num_neg)
        i = np.arange(N, dtype=np.int32)[:, None]
        _consts_cache[key_] = rand + (rand >= i).astype(np.int32)
    return _consts_cache[key_]


def _s1_body(fq_ref, fk_ref, m_ref, pos_ref):
    i = pl.program_id(0)
    qT = fq_ref[0]
    qss = jnp.sum(qT * qT, axis=0, keepdims=True)
    qnT = (qT / jnp.maximum(jnp.sqrt(qss), 1e-12)).astype(jnp.bfloat16)
    C, BQ = qT.shape
    HW = fk_ref.shape[2]
    NT = (fk_ref.shape[0] * HW) // 128
    acc_pos = jnp.zeros((BQ,), jnp.float32)
    row = lax.broadcasted_iota(jnp.int32, (BQ, 128), 0) + i * BQ
    for t in range(NT):
        b2 = (t * 128) // HW
        o = (t * 128) % HW
        k_t = fk_ref[b2][:, o:o + 128]
        kss = jnp.sum(k_t * k_t, axis=0, keepdims=True)
        kn_t = (k_t / jnp.maximum(jnp.sqrt(kss), 1e-12)).astype(jnp.bfloat16)
        m_t = lax.dot_general(qnT, kn_t, (((0,), (0,)), ((), ())),
                              preferred_element_type=jnp.float32)
        m_ref[t * BQ:(t + 1) * BQ, :] = m_t.astype(jnp.bfloat16)
        col = lax.broadcasted_iota(jnp.int32, (BQ, 128), 1) + t * 128
        acc_pos = acc_pos + jnp.sum(jnp.where(col == row, m_t, 0.0), axis=1)
    pos_ref[...] = acc_pos


def _similarity(fq3, fk3, bq=512):
    B, C, HW = fq3.shape
    N = B * HW
    nt = N // 128
    nb = HW // bq
    return pl.pallas_call(
        _s1_body,
        grid=(N // bq,),
        in_specs=[
            pl.BlockSpec((1, C, bq), lambda i: (i // nb, 0, i % nb)),
            pl.BlockSpec((B, C, HW), lambda i: (0, 0, 0)),
        ],
        out_specs=[
            pl.BlockSpec((bq * nt, 128), lambda i: (i, 0)),
            pl.BlockSpec((bq,), lambda i: (i,)),
        ],
        out_shape=[
            jax.ShapeDtypeStruct((N * nt, 128), jnp.bfloat16),
            jax.ShapeDtypeStruct((N,), jnp.float32),
        ],
    )(fq3, fk3)


def _chunk_row_constants(N, num_neg, bq):
    key_ = ("crow", N, num_neg, bq)
    if key_ not in _consts_cache:
        nt = N // 128
        hb = bq // 2
        p = np.arange(N // 2, dtype=np.int32)[:, None]
        t = np.arange(nt, dtype=np.int32)[None, :]
        crow = (p // hb) * (hb * nt) + t * hb + (p % hb)
        cidx = _negative_indices(N, num_neg)
        _consts_cache[key_] = (crow, cidx.reshape(-1))
    return _consts_cache[key_]


def _sc_neg_expsum(m_out, N, num_neg, bq):
    rows_per_w = N // _NW
    nt = N // 128
    pairs_per_w = rows_per_w // 2
    pb = 4
    nblk = pairs_per_w // pb
    crow, cidx_flat = _chunk_row_constants(N, num_neg, bq)

    mesh = plsc.VectorSubcoreMesh(core_axis_name="c", subcore_axis_name="s")

    @functools.partial(
        pl.kernel,
        mesh=mesh,
        compiler_params=pltpu.CompilerParams(needs_layout_passes=False),
        out_type=jax.ShapeDtypeStruct((N * 16,), jnp.float32),
        scratch_types=[
            pltpu.VMEM((pairs_per_w, nt), jnp.int32),
            pltpu.VMEM((rows_per_w * num_neg,), jnp.int32),
            pltpu.VMEM((pb * nt, 128), jnp.int32),
            pltpu.VMEM((pb * nt, 128), jnp.int32),
            pltpu.VMEM((rows_per_w * 16,), jnp.float32),
            pltpu.SemaphoreType.DMA,
            pltpu.SemaphoreType.DMA,
        ],
    )
    def sc_k(tab_hbm, crow_hbm, cidx_hbm, out_hbm,
             crow_v, cidx_v, buf_a, buf_b, out_v, sem_a, sem_b):
        wid = lax.axis_index("s") * _NC + lax.axis_index("c")
        tab32 = tab_hbm.bitcast(jnp.int32)
        pltpu.sync_copy(crow_hbm.at[pl.ds(wid * pairs_per_w, pairs_per_w)],
                        crow_v)
        pltpu.sync_copy(cidx_hbm.at[pl.ds(wid * rows_per_w * num_neg,
                                          rows_per_w * num_neg)], cidx_v)

        bufs = (buf_a, buf_b)
        sems = (sem_a, sem_b)

        def fire(blk):
            buf, sem = bufs[blk % 2], sems[blk % 2]
            cps = []
            for i in range(pb):
                r = blk * pb + i
                cps.append(pltpu.async_copy(
                    tab32.at[crow_v.at[r]],
                    buf.at[pl.ds(i * nt, nt)], sem))
            return cps

        def expv(buf, rows, cols, shift):
            bits = plsc.load_gather(buf, [rows, cols & 127])
            v16 = (bits >> shift) & jnp.int32(0xFFFF)
            f = plsc.bitcast(v16 << 16, jnp.float32)
            return jnp.exp(f * (1.0 / _TEMP))

        def compute(blk):
            buf = bufs[blk % 2]

            def row_body(q_loc, carry):
                pair_slot = q_loc >> 1
                shift = (q_loc & 1) * 16
                n_glob = wid * rows_per_w + blk * 2 * pb + q_loc
                dcols = jnp.full((16,), 0, jnp.int32) + n_glob
                lane0 = lax.iota(jnp.int32, 16) == 0
                de = expv(buf, (dcols >> 7) + pair_slot * nt, dcols, shift)
                acc = jnp.where(lane0, de, 0.0)
                base = pl.multiple_of(
                    (blk * 2 * pb + q_loc) * num_neg, num_neg)
                for g in range(num_neg // 16):
                    cols = cidx_v[pl.ds(base + g * 16, 16)]
                    acc = acc + expv(
                        buf, (cols >> 7) + pair_slot * nt, cols, shift)
                ooff = pl.multiple_of((blk * 2 * pb + q_loc) * 16, 16)
                out_v[pl.ds(ooff, 16)] = acc
                return carry

            lax.fori_loop(0, 2 * pb, row_body, 0)

        pending = fire(0)
        for blk in range(nblk):
            nxt = fire(blk + 1) if blk + 1 < nblk else []
            for cp in pending:
                cp.wait()
            compute(blk)
            pending = nxt

        pltpu.sync_copy(out_v, out_hbm.at[pl.ds(wid * rows_per_w * 16,
                                                rows_per_w * 16)])

    return sc_k(m_out, jnp.asarray(crow), jnp.asarray(cidx_flat))


def _s3_body(N, pos_ref, p_ref, out_ref):
    p = p_ref[...]
    gr = lax.broadcasted_iota(jnp.int32, (128, 8), 0) // 16
    gc = lax.broadcasted_iota(jnp.int32, (128, 8), 1)
    G = jnp.where(gr == gc, 1.0, 0.0)
    s8 = jnp.dot(p, G, preferred_element_type=jnp.float32)
    ts = jnp.sum(pos_ref[...]) * (1.0 / _TEMP)
    out_ref[...] = jnp.reshape((jnp.sum(jnp.log(s8)) - ts) / N, (1, 1))


def _finalize(pos32, p512, N):
    return pl.pallas_call(
        functools.partial(_s3_body, N),
        in_specs=[
            pl.BlockSpec(pos32.shape, lambda: (0, 0)),
            pl.BlockSpec(p512.shape, lambda: (0, 0)),
        ],
        out_specs=pl.BlockSpec((1, 1), lambda: (0, 0)),
        out_shape=jax.ShapeDtypeStruct((1, 1), jnp.float32),
    )(pos32, p512)


def kernel(feat_q, feat_k):
    B, C, H, W = feat_q.shape
    HW = H * W
    N = B * HW
    fq3 = feat_q.reshape(B, C, HW)
    fk3 = feat_k.reshape(B, C, HW)

    bq = 512
    m_out, pos = _similarity(fq3, fk3, bq=bq)
    partial_flat = _sc_neg_expsum(m_out, N, _NUM_NEG, bq)
    loss = _finalize(pos.reshape(N // 128, 128),
                     partial_flat.reshape(N // 8, 128), N)
    return loss[0, 0]


# speedup vs baseline: 26.1713x; 1.0107x over previous
import functools

import numpy as np
import jax
import jax.numpy as jnp
from jax import lax
from jax.experimental import pallas as pl
from jax.experimental.pallas import tpu as pltpu
from jax.experimental.pallas import tpu_sc as plsc

_TEMP = 0.07
_NUM_NEG = 128

_NC = 2
_NS = 16
_NW = _NC * _NS

_consts_cache = {}


def _rotl(x, d):
    return ((x << np.uint32(d)) | (x >> np.uint32(32 - d))).astype(np.uint32)


def _threefry2x32(k0, k1, x0, x1):
    rot = [13, 15, 26, 6, 17, 29, 16, 24]
    ks0, ks1 = np.uint32(k0), np.uint32(k1)
    ks2 = np.uint32(ks0 ^ ks1 ^ np.uint32(0x1BD11BDA))
    ks = [ks0, ks1, ks2]
    x0 = (x0 + ks0).astype(np.uint32)
    x1 = (x1 + ks1).astype(np.uint32)
    for i in range(5):
        for r in rot[(i % 2) * 4:(i % 2) * 4 + 4]:
            x0 = (x0 + x1).astype(np.uint32)
            x1 = (_rotl(x1, r) ^ x0).astype(np.uint32)
        x0 = (x0 + ks[(i + 1) % 3]).astype(np.uint32)
        x1 = (x1 + ks[(i + 2) % 3] + np.uint32(i + 1)).astype(np.uint32)
    return x0, x1


def _bits(k0, k1, size):
    counts = np.arange(size, dtype=np.uint32)
    h0, h1 = _threefry2x32(k0, k1, np.zeros(size, np.uint32), counts)
    return h0 ^ h1


def _negative_indices(N, num_neg):
    key_ = (N, num_neg)
    if key_ not in _consts_cache:
        f0, f1 = _threefry2x32(np.uint32(0), np.uint32(0),
                               np.array([0], np.uint32),
                               np.array([123], np.uint32))
        s0, s1 = _threefry2x32(f0[0], f1[0], np.zeros(2, np.uint32),
                               np.arange(2, dtype=np.uint32))
        u = _bits(s0[0], s1[0], N * num_neg)
        v = _bits(s0[1], s1[1], N * num_neg)
        span = np.uint64(N - 1)
        mult = np.uint64((np.uint64(65536 % span) ** 2) % span)
        r = ((u % span).astype(np.uint64) * mult
             + (v % span).astype(np.uint64)) % span
        rand = r.astype(np.int32).reshape(N, num_neg)
        i = np.arange(N, dtype=np.int32)[:, None]
        _consts_cache[key_] = rand + (rand >= i).astype(np.int32)
    return _consts_cache[key_]


def _s1_body(fq_ref, fk_ref, m_ref, pos_ref):
    i = pl.program_id(0)
    qT = fq_ref[0]
    qss = jnp.sum(qT * qT, axis=0, keepdims=True)
    qnT = (qT / jnp.maximum(jnp.sqrt(qss), 1e-12)).astype(jnp.bfloat16)
    C, BQ = qT.shape
    HW = fk_ref.shape[2]
    NT = (fk_ref.shape[0] * HW) // 128
    for t in range(NT):
        b2 = (t * 128) // HW
        o = (t * 128) % HW
        k_t = fk_ref[b2][:, o:o + 128]
        kss = jnp.sum(k_t * k_t, axis=0, keepdims=True)
        kn_t = (k_t / jnp.maximum(jnp.sqrt(kss), 1e-12)).astype(jnp.bfloat16)
        m_t = lax.dot_general(qnT, kn_t, (((0,), (0,)), ((), ())),
                              preferred_element_type=jnp.float32)
        m_ref[t * BQ:(t + 1) * BQ, :] = m_t.astype(jnp.bfloat16)
    eye = (lax.broadcasted_iota(jnp.int32, (128, 128), 0)
           == lax.broadcasted_iota(jnp.int32, (128, 128), 1))
    for j in range(BQ // 128):
        t_dyn = i * (BQ // 128) + j
        sq = m_ref[pl.ds(t_dyn * BQ + j * 128, 128), :].astype(jnp.float32)
        pos_ref[pl.ds(j * 128, 128)] = jnp.sum(
            jnp.where(eye, sq, 0.0), axis=1)


def _similarity(fq3, fk3, bq=512):
    B, C, HW = fq3.shape
    N = B * HW
    nt = N // 128
    nb = HW // bq
    return pl.pallas_call(
        _s1_body,
        grid=(N // bq,),
        in_specs=[
            pl.BlockSpec((1, C, bq), lambda i: (i // nb, 0, i % nb)),
            pl.BlockSpec((B, C, HW), lambda i: (0, 0, 0)),
        ],
        out_specs=[
            pl.BlockSpec((bq * nt, 128), lambda i: (i, 0)),
            pl.BlockSpec((bq,), lambda i: (i,)),
        ],
        out_shape=[
            jax.ShapeDtypeStruct((N * nt, 128), jnp.bfloat16),
            jax.ShapeDtypeStruct((N,), jnp.float32),
        ],
    )(fq3, fk3)


def _chunk_row_constants(N, num_neg, bq):
    key_ = ("crow", N, num_neg, bq)
    if key_ not in _consts_cache:
        nt = N // 128
        hb = bq // 2
        p = np.arange(N // 2, dtype=np.int32)[:, None]
        t = np.arange(nt, dtype=np.int32)[None, :]
        crow = (p // hb) * (hb * nt) + t * hb + (p % hb)
        cidx = _negative_indices(N, num_neg)
        _consts_cache[key_] = (crow, cidx.reshape(-1))
    return _consts_cache[key_]


def _sc_neg_expsum(m_out, N, num_neg, bq):
    rows_per_w = N // _NW
    nt = N // 128
    pairs_per_w = rows_per_w // 2
    pb = 4
    nblk = pairs_per_w // pb
    crow, cidx_flat = _chunk_row_constants(N, num_neg, bq)

    mesh = plsc.VectorSubcoreMesh(core_axis_name="c", subcore_axis_name="s")

    @functools.partial(
        pl.kernel,
        mesh=mesh,
        compiler_params=pltpu.CompilerParams(needs_layout_passes=False),
        out_type=jax.ShapeDtypeStruct((N * 16,), jnp.float32),
        scratch_types=[
            pltpu.VMEM((pairs_per_w, nt), jnp.int32),
            pltpu.VMEM((rows_per_w * num_neg,), jnp.int32),
            pltpu.VMEM((pb * nt, 128), jnp.int32),
            pltpu.VMEM((pb * nt, 128), jnp.int32),
            pltpu.VMEM((rows_per_w * 16,), jnp.float32),
            pltpu.SemaphoreType.DMA,
            pltpu.SemaphoreType.DMA,
        ],
    )
    def sc_k(tab_hbm, crow_hbm, cidx_hbm, out_hbm,
             crow_v, cidx_v, buf_a, buf_b, out_v, sem_a, sem_b):
        wid = lax.axis_index("s") * _NC + lax.axis_index("c")
        tab32 = tab_hbm.bitcast(jnp.int32)
        pltpu.sync_copy(crow_hbm.at[pl.ds(wid * pairs_per_w, pairs_per_w)],
                        crow_v)
        pltpu.sync_copy(cidx_hbm.at[pl.ds(wid * rows_per_w * num_neg,
                                          rows_per_w * num_neg)], cidx_v)

        bufs = (buf_a, buf_b)
        sems = (sem_a, sem_b)

        def fire(blk):
            buf, sem = bufs[blk % 2], sems[blk % 2]
            cps = []
            for i in range(pb):
                r = blk * pb + i
                cps.append(pltpu.async_copy(
                    tab32.at[crow_v.at[r]],
                    buf.at[pl.ds(i * nt, nt)], sem))
            return cps

        def expv(buf, rows, cols, shift):
            bits = plsc.load_gather(buf, [rows, cols & 127])
            v16 = (bits >> shift) & jnp.int32(0xFFFF)
            f = plsc.bitcast(v16 << 16, jnp.float32)
            return jnp.exp(f * (1.0 / _TEMP))

        def compute(blk):
            buf = bufs[blk % 2]

            def row_body(q_loc, carry):
                pair_slot = q_loc >> 1
                shift = (q_loc & 1) * 16
                n_glob = wid * rows_per_w + blk * 2 * pb + q_loc
                dcols = jnp.full((16,), 0, jnp.int32) + n_glob
                lane0 = lax.iota(jnp.int32, 16) == 0
                de = expv(buf, (dcols >> 7) + pair_slot * nt, dcols, shift)
                acc = jnp.where(lane0, de, 0.0)
                base = pl.multiple_of(
                    (blk * 2 * pb + q_loc) * num_neg, num_neg)
                for g in range(num_neg // 16):
                    cols = cidx_v[pl.ds(base + g * 16, 16)]
                    acc = acc + expv(
                        buf, (cols >> 7) + pair_slot * nt, cols, shift)
                ooff = pl.multiple_of((blk * 2 * pb + q_loc) * 16, 16)
                out_v[pl.ds(ooff, 16)] = acc
                return carry

            lax.fori_loop(0, 2 * pb, row_body, 0)

        pending = fire(0)
        for blk in range(nblk):
            nxt = fire(blk + 1) if blk + 1 < nblk else []
            for cp in pending:
                cp.wait()
            compute(blk)
            pending = nxt

        pltpu.sync_copy(out_v, out_hbm.at[pl.ds(wid * rows_per_w * 16,
                                                rows_per_w * 16)])

    return sc_k(m_out, jnp.asarray(crow), jnp.asarray(cidx_flat))


def _s3_body(N, pos_ref, p_ref, out_ref):
    p = p_ref[...]
    gr = lax.broadcasted_iota(jnp.int32, (128, 8), 0) // 16
    gc = lax.broadcasted_iota(jnp.int32, (128, 8), 1)
    G = jnp.where(gr == gc, 1.0, 0.0)
    s8 = jnp.dot(p, G, preferred_element_type=jnp.float32)
    ts = jnp.sum(pos_ref[...]) * (1.0 / _TEMP)
    out_ref[...] = jnp.reshape((jnp.sum(jnp.log(s8)) - ts) / N, (1, 1))


def _finalize(pos32, p512, N):
    return pl.pallas_call(
        functools.partial(_s3_body, N),
        in_specs=[
            pl.BlockSpec(pos32.shape, lambda: (0, 0)),
            pl.BlockSpec(p512.shape, lambda: (0, 0)),
        ],
        out_specs=pl.BlockSpec((1, 1), lambda: (0, 0)),
        out_shape=jax.ShapeDtypeStruct((1, 1), jnp.float32),
    )(pos32, p512)


def kernel(feat_q, feat_k):
    B, C, H, W = feat_q.shape
    HW = H * W
    N = B * HW
    fq3 = feat_q.reshape(B, C, HW)
    fk3 = feat_k.reshape(B, C, HW)

    bq = 512
    m_out, pos = _similarity(fq3, fk3, bq=bq)
    partial_flat = _sc_neg_expsum(m_out, N, _NUM_NEG, bq)
    loss = _finalize(pos.reshape(N // 128, 128),
                     partial_flat.reshape(N // 8, 128), N)
    return loss[0, 0]


# speedup vs baseline: 28.7833x; 1.0998x over previous
import functools

import numpy as np
import jax
import jax.numpy as jnp
from jax import lax
from jax.experimental import pallas as pl
from jax.experimental.pallas import tpu as pltpu
from jax.experimental.pallas import tpu_sc as plsc

_TEMP = 0.07
_NUM_NEG = 128

_NC = 2
_NS = 16
_NW = _NC * _NS

_consts_cache = {}


def _rotl(x, d):
    return ((x << np.uint32(d)) | (x >> np.uint32(32 - d))).astype(np.uint32)


def _threefry2x32(k0, k1, x0, x1):
    rot = [13, 15, 26, 6, 17, 29, 16, 24]
    ks0, ks1 = np.uint32(k0), np.uint32(k1)
    ks2 = np.uint32(ks0 ^ ks1 ^ np.uint32(0x1BD11BDA))
    ks = [ks0, ks1, ks2]
    x0 = (x0 + ks0).astype(np.uint32)
    x1 = (x1 + ks1).astype(np.uint32)
    for i in range(5):
        for r in rot[(i % 2) * 4:(i % 2) * 4 + 4]:
            x0 = (x0 + x1).astype(np.uint32)
            x1 = (_rotl(x1, r) ^ x0).astype(np.uint32)
        x0 = (x0 + ks[(i + 1) % 3]).astype(np.uint32)
        x1 = (x1 + ks[(i + 2) % 3] + np.uint32(i + 1)).astype(np.uint32)
    return x0, x1


def _bits(k0, k1, size):
    counts = np.arange(size, dtype=np.uint32)
    h0, h1 = _threefry2x32(k0, k1, np.zeros(size, np.uint32), counts)
    return h0 ^ h1


def _negative_indices(N, num_neg):
    key_ = (N, num_neg)
    if key_ not in _consts_cache:
        f0, f1 = _threefry2x32(np.uint32(0), np.uint32(0),
                               np.array([0], np.uint32),
                               np.array([123], np.uint32))
        s0, s1 = _threefry2x32(f0[0], f1[0], np.zeros(2, np.uint32),
                               np.arange(2, dtype=np.uint32))
        u = _bits(s0[0], s1[0], N * num_neg)
        v = _bits(s0[1], s1[1], N * num_neg)
        span = np.uint64(N - 1)
        mult = np.uint64((np.uint64(65536 % span) ** 2) % span)
        r = ((u % span).astype(np.uint64) * mult
             + (v % span).astype(np.uint64)) % span
        rand = r.astype(np.int32).reshape(N, num_neg)
        i = np.arange(N, dtype=np.int32)[:, None]
        _consts_cache[key_] = rand + (rand >= i).astype(np.int32)
    return _consts_cache[key_]


def _s1_body(fq_ref, fk_ref, m_ref, pos_ref):
    i = pl.program_id(0)
    qT = fq_ref[0]
    qss = jnp.sum(qT * qT, axis=0, keepdims=True)
    qnT = (qT / jnp.maximum(jnp.sqrt(qss), 1e-12)).astype(jnp.bfloat16)
    C, BQ = qT.shape
    HW = fk_ref.shape[2]
    NT = (fk_ref.shape[0] * HW) // 128
    for t2 in range(NT // 2):
        b2 = (t2 * 256) // HW
        o = (t2 * 256) % HW
        k_t = fk_ref[b2][:, o:o + 256]
        kss = jnp.sum(k_t * k_t, axis=0, keepdims=True)
        kn_t = (k_t / jnp.maximum(jnp.sqrt(kss), 1e-12)).astype(jnp.bfloat16)
        m_t = lax.dot_general(qnT, kn_t, (((0,), (0,)), ((), ())),
                              preferred_element_type=jnp.float32)
        mbf = m_t.astype(jnp.bfloat16)
        m_ref[(2 * t2) * BQ:(2 * t2 + 1) * BQ, :] = mbf[:, :128]
        m_ref[(2 * t2 + 1) * BQ:(2 * t2 + 2) * BQ, :] = mbf[:, 128:]
    eye = (lax.broadcasted_iota(jnp.int32, (128, 128), 0)
           == lax.broadcasted_iota(jnp.int32, (128, 128), 1))
    for j in range(BQ // 128):
        t_dyn = i * (BQ // 128) + j
        sq = m_ref[pl.ds(t_dyn * BQ + j * 128, 128), :].astype(jnp.float32)
        pos_ref[pl.ds(j * 128, 128)] = jnp.sum(
            jnp.where(eye, sq, 0.0), axis=1)


def _similarity(fq3, fk3, bq=512):
    B, C, HW = fq3.shape
    N = B * HW
    nt = N // 128
    nb = HW // bq
    return pl.pallas_call(
        _s1_body,
        grid=(N // bq,),
        in_specs=[
            pl.BlockSpec((1, C, bq), lambda i: (i // nb, 0, i % nb)),
            pl.BlockSpec((B, C, HW), lambda i: (0, 0, 0)),
        ],
        out_specs=[
            pl.BlockSpec((bq * nt, 128), lambda i: (i, 0)),
            pl.BlockSpec((bq,), lambda i: (i,)),
        ],
        out_shape=[
            jax.ShapeDtypeStruct((N * nt, 128), jnp.bfloat16),
            jax.ShapeDtypeStruct((N,), jnp.float32),
        ],
    )(fq3, fk3)


def _chunk_row_constants(N, num_neg, bq):
    key_ = ("crow", N, num_neg, bq)
    if key_ not in _consts_cache:
        nt = N // 128
        hb = bq // 2
        p = np.arange(N // 2, dtype=np.int32)[:, None]
        t = np.arange(nt, dtype=np.int32)[None, :]
        crow = (p // hb) * (hb * nt) + t * hb + (p % hb)
        cidx = _negative_indices(N, num_neg)
        _consts_cache[key_] = (crow, cidx.reshape(-1))
    return _consts_cache[key_]


def _sc_neg_expsum(m_out, N, num_neg, bq):
    rows_per_w = N // _NW
    nt = N // 128
    pairs_per_w = rows_per_w // 2
    pb = 4
    nblk = pairs_per_w // pb
    crow, cidx_flat = _chunk_row_constants(N, num_neg, bq)

    mesh = plsc.VectorSubcoreMesh(core_axis_name="c", subcore_axis_name="s")

    @functools.partial(
        pl.kernel,
        mesh=mesh,
        compiler_params=pltpu.CompilerParams(needs_layout_passes=False),
        out_type=jax.ShapeDtypeStruct((N * 16,), jnp.float32),
        scratch_types=[
            pltpu.VMEM((pairs_per_w, nt), jnp.int32),
            pltpu.VMEM((rows_per_w * num_neg,), jnp.int32),
            pltpu.VMEM((pb * nt, 128), jnp.int32),
            pltpu.VMEM((pb * nt, 128), jnp.int32),
            pltpu.VMEM((rows_per_w * 16,), jnp.float32),
            pltpu.SemaphoreType.DMA,
            pltpu.SemaphoreType.DMA,
        ],
    )
    def sc_k(tab_hbm, crow_hbm, cidx_hbm, out_hbm,
             crow_v, cidx_v, buf_a, buf_b, out_v, sem_a, sem_b):
        wid = lax.axis_index("s") * _NC + lax.axis_index("c")
        tab32 = tab_hbm.bitcast(jnp.int32)
        pltpu.sync_copy(crow_hbm.at[pl.ds(wid * pairs_per_w, pairs_per_w)],
                        crow_v)
        pltpu.sync_copy(cidx_hbm.at[pl.ds(wid * rows_per_w * num_neg,
                                          rows_per_w * num_neg)], cidx_v)

        bufs = (buf_a, buf_b)
        sems = (sem_a, sem_b)

        def fire(blk):
            buf, sem = bufs[blk % 2], sems[blk % 2]
            cps = []
            for i in range(pb):
                r = blk * pb + i
                cps.append(pltpu.async_copy(
                    tab32.at[crow_v.at[r]],
                    buf.at[pl.ds(i * nt, nt)], sem))
            return cps

        def expv(buf, rows, cols, shift):
            bits = plsc.load_gather(buf, [rows, cols & 127])
            v16 = (bits >> shift) & jnp.int32(0xFFFF)
            f = plsc.bitcast(v16 << 16, jnp.float32)
            return jnp.exp(f * (1.0 / _TEMP))

        def compute(blk):
            buf = bufs[blk % 2]

            def row_body(q_loc, carry):
                pair_slot = q_loc >> 1
                shift = (q_loc & 1) * 16
                n_glob = wid * rows_per_w + blk * 2 * pb + q_loc
                dcols = jnp.full((16,), 0, jnp.int32) + n_glob
                lane0 = lax.iota(jnp.int32, 16) == 0
                de = expv(buf, (dcols >> 7) + pair_slot * nt, dcols, shift)
                acc = jnp.where(lane0, de, 0.0)
                base = pl.multiple_of(
                    (blk * 2 * pb + q_loc) * num_neg, num_neg)
                for g in range(num_neg // 16):
                    cols = cidx_v[pl.ds(base + g * 16, 16)]
                    acc = acc + expv(
                        buf, (cols >> 7) + pair_slot * nt, cols, shift)
                ooff = pl.multiple_of((blk * 2 * pb + q_loc) * 16, 16)
                out_v[pl.ds(ooff, 16)] = acc
                return carry

            lax.fori_loop(0, 2 * pb, row_body, 0)

        pending = fire(0)
        for blk in range(nblk):
            nxt = fire(blk + 1) if blk + 1 < nblk else []
            for cp in pending:
                cp.wait()
            compute(blk)
            pending = nxt

        pltpu.sync_copy(out_v, out_hbm.at[pl.ds(wid * rows_per_w * 16,
                                                rows_per_w * 16)])

    return sc_k(m_out, jnp.asarray(crow), jnp.asarray(cidx_flat))


def _s3_body(N, pos_ref, p_ref, out_ref):
    p = p_ref[...]
    gr = lax.broadcasted_iota(jnp.int32, (128, 8), 0) // 16
    gc = lax.broadcasted_iota(jnp.int32, (128, 8), 1)
    G = jnp.where(gr == gc, 1.0, 0.0)
    s8 = jnp.dot(p, G, preferred_element_type=jnp.float32)
    ts = jnp.sum(pos_ref[...]) * (1.0 / _TEMP)
    out_ref[...] = jnp.reshape((jnp.sum(jnp.log(s8)) - ts) / N, (1, 1))


def _finalize(pos32, p512, N):
    return pl.pallas_call(
        functools.partial(_s3_body, N),
        in_specs=[
            pl.BlockSpec(pos32.shape, lambda: (0, 0)),
            pl.BlockSpec(p512.shape, lambda: (0, 0)),
        ],
        out_specs=pl.BlockSpec((1, 1), lambda: (0, 0)),
        out_shape=jax.ShapeDtypeStruct((1, 1), jnp.float32),
    )(pos32, p512)


def kernel(feat_q, feat_k):
    B, C, H, W = feat_q.shape
    HW = H * W
    N = B * HW
    fq3 = feat_q.reshape(B, C, HW)
    fk3 = feat_k.reshape(B, C, HW)

    bq = 512
    m_out, pos = _similarity(fq3, fk3, bq=bq)
    partial_flat = _sc_neg_expsum(m_out, N, _NUM_NEG, bq)
    loss = _finalize(pos.reshape(N // 128, 128),
                     partial_flat.reshape(N // 8, 128), N)
    return loss[0, 0]
